# Initial kernel scaffold; baseline (speedup 1.0000x reference)
import jax
import jax.numpy as jnp
from jax.experimental import pallas as pl


def kernel(x, edge_index, W1, b1, W2, b2):
    raise NotImplementedError("write your pallas kernel here")


# speedup vs baseline: 8.5572x; 8.5572x over previous
import functools

import jax
import jax.numpy as jnp
from jax import lax
from jax.experimental import pallas as pl
from jax.experimental.pallas import tpu as pltpu
from jax.experimental.pallas import tpu_sc as plsc

N = 10000
NPAD = 10240
PADIDX = 10000
E = 320000
EPAD = 327680
NW = 16
CHUNKS = 160
K = 128
STRIPE = NPAD // NW

PH = 2
PH_ROWS = NPAD // PH
ACC_ROWS = PH_ROWS + 256
ASTRIPE = ACC_ROWS // NW
PSTRIPE = PH_ROWS // NW

_mesh = plsc.VectorSubcoreMesh(
    core_axis_name="c", subcore_axis_name="s", num_cores=1
)
_f32 = jnp.float32
_i32 = jnp.int32
_sc_params = pltpu.CompilerParams(needs_layout_passes=False)


def _hist_update(hist, tmp, keys, vals, iota):
    ks, vs = plsc.sort_key_val(keys, vals)
    cs = plsc.cumsum(vs)
    tmp[...] = ks
    knext = plsc.load_gather(tmp, [jnp.minimum(iota + 1, 15)])
    mend = (ks != knext) | (iota == 15)
    plsc.addupdate_scatter(hist, [ks], cs, mask=mend)
    plsc.addupdate_scatter(hist, [knext], -cs, mask=mend & (iota < 15))


NVEC = NPAD // 16


@functools.partial(
    pl.kernel,
    out_type=jax.ShapeDtypeStruct((NW, NPAD), _f32),
    mesh=_mesh,
    compiler_params=_sc_params,
    scratch_types=[
        pltpu.VMEM((CHUNKS * K,), jnp.int32),
        pltpu.VMEM((NPAD,), _f32),
        pltpu.VMEM((16,), jnp.int32),
    ],
)
def _deg_kernel(dst_hbm, degh_out, idx_v, hist, tmp):
    sid = lax.axis_index("s")
    pltpu.sync_copy(dst_hbm.at[sid], idx_v)

    zero16 = jnp.zeros((16,), _f32)

    def zbody(i, _):
        hist[pl.ds(pl.multiple_of(i * 16, 16), 16)] = zero16
        return ()

    lax.fori_loop(0, NVEC, zbody, ())

    iota = lax.iota(_i32, 16)
    cnt = jnp.ones((16,), _f32)

    def cbody(j, _):
        for c in range(K // 16):
            k16 = idx_v[pl.ds(pl.multiple_of(j * K + c * 16, 16), 16)]
            _hist_update(hist, tmp, k16, cnt, iota)
        return ()

    lax.fori_loop(0, CHUNKS, cbody, ())
    pltpu.sync_copy(hist, degh_out.at[sid])


def _scale_body(xp_ref, degh_ref, xs_ref, dv_ref):
    deg = jnp.sum(degh_ref[...], axis=0) + 1.0
    dinv = lax.rsqrt(deg)
    dv_ref[...] = dinv[None, :]
    xs_ref[...] = xp_ref[...] * dinv[:, None]


_scale = pl.pallas_call(
    _scale_body,
    out_shape=(
        jax.ShapeDtypeStruct((NPAD, 128), _f32),
        jax.ShapeDtypeStruct((1, NPAD), _f32),
    ),
)


@functools.partial(
    pl.kernel,
    out_type=jax.ShapeDtypeStruct((NPAD, 128), _f32),
    mesh=_mesh,
    scratch_types=[
        pltpu.VMEM((CHUNKS, K), jnp.int32),
        pltpu.VMEM((CHUNKS, K), jnp.int32),
        pltpu.VMEM((CHUNKS, K), jnp.int32),
        pltpu.VMEM((K, 128), _f32),
        pltpu.VMEM((16, 128), _f32),
        pltpu.VMEM_SHARED((ACC_ROWS, 128), _f32),
    ],
)
def _scatter_kernel(xs_hbm, src_hbm, dst_hbm, p0_out,
                    sidx_v, didx_v, ridx_v, xb, zacc, acc_sh):
    sid = lax.axis_index("s")

    zero16 = jnp.zeros((16,), _f32)
    for i in range(16):
        for c in range(8):
            zacc[i, pl.ds(c * 16, 16)] = zero16

    pltpu.sync_copy(src_hbm.at[sid], sidx_v)
    pltpu.sync_copy(dst_hbm.at[sid], didx_v)

    for ph in range(PH):
        for t in range(ASTRIPE // 16):
            pltpu.sync_copy(
                zacc, acc_sh.at[pl.ds(sid * ASTRIPE + t * 16, 16)]
            )
        if ASTRIPE % 16:
            pltpu.sync_copy(
                zacc.at[pl.ds(0, ASTRIPE % 16)],
                acc_sh.at[pl.ds(sid * ASTRIPE + (ASTRIPE // 16) * 16,
                                ASTRIPE % 16)],
            )

        def rbody(j, _):
            for c in range(K // 16):
                d16 = didx_v[j, pl.ds(c * 16, 16)]
                loc = d16 - (ph * PH_ROWS)
                ok = (loc >= 0) & (loc < PH_ROWS)
                dummy = PH_ROWS + jnp.bitwise_and(d16, 63)
                ridx_v[j, pl.ds(c * 16, 16)] = jnp.where(ok, loc, dummy)
            return ()

        lax.fori_loop(0, CHUNKS, rbody, ())
        plsc.subcore_barrier()

        def sbody(j, _):
            pltpu.sync_copy(xs_hbm.at[sidx_v.at[j]], xb)
            pltpu.sync_copy(xb, acc_sh.at[ridx_v.at[j]], add=True)
            return ()

        lax.fori_loop(0, CHUNKS, sbody, ())
        plsc.subcore_barrier()

        pltpu.sync_copy(
            acc_sh.at[pl.ds(sid * PSTRIPE, PSTRIPE)],
            p0_out.at[pl.ds(ph * PH_ROWS + sid * PSTRIPE, PSTRIPE)],
        )
        plsc.subcore_barrier()


@functools.partial(
    pl.kernel,
    out_type=jax.ShapeDtypeStruct((NW, NPAD), _f32),
    mesh=_mesh,
    compiler_params=_sc_params,
    scratch_types=[
        pltpu.VMEM((CHUNKS * K,), jnp.int32),
        pltpu.VMEM((CHUNKS * K,), jnp.int32),
        pltpu.VMEM((NPAD,), _f32),
        pltpu.VMEM((NPAD,), _f32),
        pltpu.VMEM((16,), jnp.int32),
    ],
)
def _wsum_kernel(src_hbm, dst_hbm, dv_hbm, wsh_out,
                 sidx_v, didx_v, dinv_v, whist, tmp):
    sid = lax.axis_index("s")

    pltpu.sync_copy(src_hbm.at[sid], sidx_v)
    pltpu.sync_copy(dst_hbm.at[sid], didx_v)
    pltpu.sync_copy(dv_hbm.at[0], dinv_v)

    zero16 = jnp.zeros((16,), _f32)

    def zbody(i, _):
        whist[pl.ds(pl.multiple_of(i * 16, 16), 16)] = zero16
        return ()

    lax.fori_loop(0, NVEC, zbody, ())

    iota = lax.iota(_i32, 16)

    def cbody(j, _):
        for c in range(K // 16):
            off = pl.multiple_of(j * K + c * 16, 16)
            d16 = didx_v[pl.ds(off, 16)]
            s16 = sidx_v[pl.ds(off, 16)]
            dval = plsc.load_gather(dinv_v, [d16])
            _hist_update(whist, tmp, s16, dval, iota)
        return ()

    lax.fori_loop(0, CHUNKS, cbody, ())
    pltpu.sync_copy(whist, wsh_out.at[sid])


def _final_body(p0_ref, xp_ref, dv_ref, wsh_ref, W1_ref, b1_ref, W2_ref,
                b2_ref, out_ref):
    dinv = dv_ref[0, :]
    dcol = dinv[:, None]
    p = dcol * p0_ref[...] + (dcol * dcol) * xp_ref[...]
    h1 = jnp.maximum(
        jnp.dot(p, W1_ref[...], preferred_element_type=_f32) + b1_ref[...],
        0.0,
    )
    ws1 = jnp.sum(wsh_ref[...], axis=0)
    w = dinv * ws1 + dinv * dinv
    rows = lax.broadcasted_iota(_i32, (NPAD, 1), 0)
    wcol = jnp.where(rows < N, w[:, None], 0.0)
    u = jnp.sum(h1 * wcol, axis=0, keepdims=True)
    out_ref[...] = (
        jnp.dot(u * (1.0 / N), W2_ref[...], preferred_element_type=_f32)
        + b2_ref[...]
    )


_final = pl.pallas_call(
    _final_body,
    out_shape=jax.ShapeDtypeStruct((1, 128), _f32),
)


def kernel(x, edge_index, W1, b1, W2, b2):
    ei = edge_index.astype(jnp.int32)
    ei = jnp.concatenate(
        [ei, jnp.full((2, EPAD - E), PADIDX, jnp.int32)], axis=1
    )
    src3 = ei[0].reshape(NW, CHUNKS, K)
    dst3 = ei[1].reshape(NW, CHUNKS, K)
    xp = jnp.pad(x, ((0, NPAD - N), (0, 0)))

    srcf = ei[0].reshape(NW, CHUNKS * K)
    dstf = ei[1].reshape(NW, CHUNKS * K)
    degh = _deg_kernel(dstf)
    xs, dv = _scale(xp, degh)
    p0 = _scatter_kernel(xs, src3, dst3)
    wsh = _wsum_kernel(srcf, dstf, dv)
    out = _final(p0, xp, dv, wsh, W1, b1.reshape(1, -1), W2,
                 b2.reshape(1, -1))
    return out.reshape(128)
